# baseline (speedup 1.0000x reference)
import functools

import jax
import jax.numpy as jnp
from jax import lax
from jax.experimental import pallas as pl
from jax.experimental.pallas import tpu as pltpu
from jax.experimental.pallas import tpu_sc as plsc

_LANES = 16
_CH = 2
_NBUF = 5
_GRP = 16


def kernel(user_table, item_table, user_ids, item_ids):
    B = user_ids.shape[0]
    D = user_table.shape[1]
    info = plsc.get_sparse_core_info()
    NC, NS = info.num_cores, info.num_subcores
    NW = NC * NS
    bpw = B // NW

    ut_t = user_table.T

    mesh = plsc.VectorSubcoreMesh(core_axis_name="c", subcore_axis_name="s")
    cparams = pltpu.CompilerParams(
        needs_layout_passes=False, use_tc_tiling_on_sc=True)

    ublk = pltpu.VMEM((_CH, D, 128), jnp.float32)
    vblk = pltpu.VMEM((_CH, 8, D), jnp.float32)

    @functools.partial(
        pl.kernel,
        mesh=mesh,
        compiler_params=cparams,
        out_type=jax.ShapeDtypeStruct((B, D), jnp.float32),
        scratch_types=[
            pltpu.VMEM((bpw,), jnp.int32),
            ublk, ublk, ublk, ublk, ublk,
            pltpu.VMEM((bpw, D), jnp.float32),
        ] + [pltpu.SemaphoreType.DMA] * _NBUF,
    )
    def user_gather(ut_hbm, uid_hbm, out_hbm, uid_v,
                    ublk0, ublk1, ublk2, ublk3, ublk4, urows_v,
                    sem_u0, sem_u1, sem_u2, sem_u3, sem_u4):
        wid = lax.axis_index("s") * NC + lax.axis_index("c")
        base = wid * bpw
        pltpu.sync_copy(uid_hbm.at[pl.ds(base, bpw)], uid_v)

        ubufs = (ublk0, ublk1, ublk2, ublk3, ublk4)
        usems = (sem_u0, sem_u1, sem_u2, sem_u3, sem_u4)
        n_chunks = bpw // _CH

        def fire(c):
            b = c % _NBUF
            g16 = ((c * _CH) // _GRP) * _GRP
            u16 = uid_v[pl.ds(g16, _GRP)]
            hs = []
            for j in range(_CH):
                lane = (c * _CH + j) % _GRP
                uc = pl.multiple_of(u16[lane] & -128, 128)
                hs.append(pltpu.async_copy(
                    ut_hbm.at[:, pl.ds(uc, 128)], ubufs[b].at[j], usems[b]))
            return hs

        pending = [fire(w) for w in range(_NBUF - 1)]
        lanes = lax.iota(jnp.int32, _LANES)
        for c in range(n_chunks):
            if c + _NBUF - 1 < n_chunks:
                pending.append(fire(c + _NBUF - 1))
            for h in pending.pop(0):
                h.wait()
            b = c % _NBUF
            g16 = ((c * _CH) // _GRP) * _GRP
            u16 = uid_v[pl.ds(g16, _GRP)]
            for j in range(_CH):
                i = c * _CH + j
                ul = u16[i % _GRP] & 127
                for q in range(D // _LANES):
                    rows = q * _LANES + lanes
                    urows_v[i, pl.ds(q * _LANES, _LANES)] = plsc.load_gather(
                        ubufs[b], [jnp.full((_LANES,), j, jnp.int32), rows,
                                   jnp.full((_LANES,), ul, jnp.int32)])

        pltpu.sync_copy(urows_v, out_hbm.at[pl.ds(base, bpw), :])

    @functools.partial(
        pl.kernel,
        mesh=mesh,
        compiler_params=cparams,
        out_type=jax.ShapeDtypeStruct((B,), jnp.float32),
        scratch_types=[
            pltpu.VMEM((bpw,), jnp.int32),
            pltpu.VMEM((bpw, D), jnp.float32),
            vblk, vblk, vblk, vblk, vblk,
            pltpu.VMEM((_GRP, _LANES), jnp.float32),
            pltpu.VMEM((bpw,), jnp.float32),
        ] + [pltpu.SemaphoreType.DMA] * _NBUF,
    )
    def item_dot(urows_hbm, it_hbm, iid_hbm, out_hbm,
                 iid_v, urows_v,
                 vblk0, vblk1, vblk2, vblk3, vblk4, pstage, acc_v,
                 sem_v0, sem_v1, sem_v2, sem_v3, sem_v4):
        wid = lax.axis_index("s") * NC + lax.axis_index("c")
        base = wid * bpw
        pltpu.sync_copy(iid_hbm.at[pl.ds(base, bpw)], iid_v)
        pltpu.sync_copy(urows_hbm.at[pl.ds(base, bpw), :], urows_v)

        vbufs = (vblk0, vblk1, vblk2, vblk3, vblk4)
        vsems = (sem_v0, sem_v1, sem_v2, sem_v3, sem_v4)
        n_chunks = bpw // _CH

        def fire(c):
            b = c % _NBUF
            g16 = ((c * _CH) // _GRP) * _GRP
            i16 = iid_v[pl.ds(g16, _GRP)]
            hs = []
            for j in range(_CH):
                lane = (c * _CH + j) % _GRP
                r8 = pl.multiple_of(i16[lane] & -8, 8)
                hs.append(pltpu.async_copy(
                    it_hbm.at[pl.ds(r8, 8), :], vbufs[b].at[j], vsems[b]))
            return hs

        pending = [fire(w) for w in range(_NBUF - 1)]
        lanes = lax.iota(jnp.int32, _LANES)
        for c in range(n_chunks):
            if c + _NBUF - 1 < n_chunks:
                pending.append(fire(c + _NBUF - 1))
            for h in pending.pop(0):
                h.wait()
            b = c % _NBUF
            g16 = ((c * _CH) // _GRP) * _GRP
            i16 = iid_v[pl.ds(g16, _GRP)]
            for j in range(_CH):
                i = c * _CH + j
                il = i16[i % _GRP] & 7
                p = jnp.zeros((_LANES,), jnp.float32)
                for q in range(D // _LANES):
                    rows = q * _LANES + lanes
                    vq = plsc.load_gather(
                        vbufs[b], [jnp.full((_LANES,), j, jnp.int32),
                                   jnp.full((_LANES,), il, jnp.int32), rows])
                    p = p + urows_v[i, pl.ds(q * _LANES, _LANES)] * vq
                pstage[i % _GRP, :] = p
            if (c * _CH + _CH) % _GRP == 0:
                acc = jnp.zeros((_LANES,), jnp.float32)
                for d in range(_LANES):
                    acc = acc + plsc.load_gather(
                        pstage, [lanes, jnp.full((_LANES,), d, jnp.int32)])
                g = (c * _CH) // _GRP
                acc_v[pl.ds(g * _GRP, _GRP)] = acc

        pltpu.sync_copy(acc_v, out_hbm.at[pl.ds(base, bpw)])

    u_rows = user_gather(ut_t, user_ids)
    return item_dot(u_rows, item_table, item_ids)


# speedup vs baseline: 1.0506x; 1.0506x over previous
import functools

import jax
import jax.numpy as jnp
from jax import lax
from jax.experimental import pallas as pl
from jax.experimental.pallas import tpu as pltpu
from jax.experimental.pallas import tpu_sc as plsc

_LANES = 16
_CH = 2
_NBUF = 5
_GRP = 16


def kernel(user_table, item_table, user_ids, item_ids):
    B = user_ids.shape[0]
    D = user_table.shape[1]
    info = plsc.get_sparse_core_info()
    NC, NS = info.num_cores, info.num_subcores
    NW = NC * NS
    bpw = B // NW

    ut_t = user_table.T

    mesh = plsc.VectorSubcoreMesh(core_axis_name="c", subcore_axis_name="s")

    ublk = pltpu.VMEM((_CH, D, 128), jnp.float32)
    vblk = pltpu.VMEM((_CH, 8, D), jnp.float32)

    @functools.partial(
        pl.kernel,
        mesh=mesh,
        compiler_params=pltpu.CompilerParams(
            needs_layout_passes=False, use_tc_tiling_on_sc=True),
        out_type=jax.ShapeDtypeStruct((B,), jnp.float32),
        scratch_types=[
            pltpu.VMEM((bpw,), jnp.int32),
            pltpu.VMEM((bpw,), jnp.int32),
            ublk, ublk, ublk, ublk, ublk, vblk, vblk, vblk, vblk, vblk,
            pltpu.VMEM((_GRP, _LANES), jnp.float32),
            pltpu.VMEM((bpw,), jnp.float32),
        ] + [pltpu.SemaphoreType.DMA] * (2 * _NBUF),
    )
    def sc_score(ut_hbm, it_hbm, uid_hbm, iid_hbm, out_hbm,
                 uid_v, iid_v,
                 ublk0, ublk1, ublk2, ublk3, ublk4,
                 vblk0, vblk1, vblk2, vblk3, vblk4,
                 pstage, acc_v,
                 sem_u0, sem_u1, sem_u2, sem_u3, sem_u4,
                 sem_v0, sem_v1, sem_v2, sem_v3, sem_v4):
        wid = lax.axis_index("s") * NC + lax.axis_index("c")
        base = wid * bpw
        pltpu.sync_copy(uid_hbm.at[pl.ds(base, bpw)], uid_v)
        pltpu.sync_copy(iid_hbm.at[pl.ds(base, bpw)], iid_v)

        ubufs = (ublk0, ublk1, ublk2, ublk3, ublk4)
        vbufs = (vblk0, vblk1, vblk2, vblk3, vblk4)
        usems = (sem_u0, sem_u1, sem_u2, sem_u3, sem_u4)
        vsems = (sem_v0, sem_v1, sem_v2, sem_v3, sem_v4)
        n_chunks = bpw // _CH

        def fire(c):
            b = c % _NBUF
            g16 = ((c * _CH) // _GRP) * _GRP
            u16 = uid_v[pl.ds(g16, _GRP)]
            i16 = iid_v[pl.ds(g16, _GRP)]
            hs = []
            for j in range(_CH):
                lane = (c * _CH + j) % _GRP
                uc = pl.multiple_of(u16[lane] & -128, 128)
                hs.append(pltpu.async_copy(
                    ut_hbm.at[:, pl.ds(uc, 128)], ubufs[b].at[j], usems[b]))
                r8 = pl.multiple_of(i16[lane] & -8, 8)
                hs.append(pltpu.async_copy(
                    it_hbm.at[pl.ds(r8, 8), :], vbufs[b].at[j], vsems[b]))
            return hs

        pending = [fire(w) for w in range(_NBUF - 1)]
        lanes = lax.iota(jnp.int32, _LANES)
        for c in range(n_chunks):
            if c + _NBUF - 1 < n_chunks:
                pending.append(fire(c + _NBUF - 1))
            for h in pending.pop(0):
                h.wait()
            b = c % _NBUF
            g16 = ((c * _CH) // _GRP) * _GRP
            u16 = uid_v[pl.ds(g16, _GRP)]
            i16 = iid_v[pl.ds(g16, _GRP)]
            for j in range(_CH):
                i = c * _CH + j
                ul = u16[i % _GRP] & 127
                il = i16[i % _GRP] & 7
                p = jnp.zeros((_LANES,), jnp.float32)
                for q in range(D // _LANES):
                    rows = q * _LANES + lanes
                    ug = plsc.load_gather(
                        ubufs[b], [jnp.full((_LANES,), j, jnp.int32), rows,
                                   jnp.full((_LANES,), ul, jnp.int32)])
                    vq = plsc.load_gather(
                        vbufs[b], [jnp.full((_LANES,), j, jnp.int32),
                                   jnp.full((_LANES,), il, jnp.int32), rows])
                    p = p + ug * vq
                pstage[i % _GRP, :] = p
            if (c * _CH + _CH) % _GRP == 0:
                acc = jnp.zeros((_LANES,), jnp.float32)
                for d in range(_LANES):
                    acc = acc + plsc.load_gather(
                        pstage, [lanes, jnp.full((_LANES,), d, jnp.int32)])
                g = (c * _CH) // _GRP
                acc_v[pl.ds(g * _GRP, _GRP)] = acc

        pltpu.sync_copy(acc_v, out_hbm.at[pl.ds(base, bpw)])

    return sc_score(ut_t, item_table, user_ids, item_ids)
